# baseline (speedup 1.0000x reference)
import functools

import jax
import jax.numpy as jnp
from jax import lax
from jax.experimental import pallas as pl
from jax.experimental.pallas import tpu as pltpu
from jax.experimental.pallas import tpu_sc as plsc

B, T, M, D = 8, 1024, 6, 256
NI = 4096
KI = 1024
NW = 32


def _img_sort_body(noise_ref, out_ref):
    keys = noise_ref[0]
    r = lax.broadcasted_iota(jnp.int32, (32, 128), 0)
    c = lax.broadcasted_iota(jnp.int32, (32, 128), 1)
    e = r * 128 + c
    idx = e
    k = 2
    while k <= NI:
        j = k // 2
        while j >= 1:
            if j < 128:
                ax, sh = 1, j
            else:
                ax, sh = 0, j // 128
            lower = (e & j) == 0
            pk = jnp.where(lower, jnp.roll(keys, -sh, axis=ax),
                           jnp.roll(keys, sh, axis=ax))
            pi = jnp.where(lower, jnp.roll(idx, -sh, axis=ax),
                           jnp.roll(idx, sh, axis=ax))
            a_first = (keys < pk) | ((keys == pk) & (idx < pi))
            want_small = lower == ((e & k) == 0)
            take_a = a_first == want_small
            keys = jnp.where(take_a, keys, pk)
            idx = jnp.where(take_a, idx, pi)
            j //= 2
        k *= 2
    out_ref[0] = idx[:8] + 1


def _img_sort(noise_img):
    x = noise_img.reshape(B, 32, 128)
    out = pl.pallas_call(
        _img_sort_body,
        grid=(B,),
        in_specs=[pl.BlockSpec((1, 32, 128), lambda b: (b, 0, 0))],
        out_specs=pl.BlockSpec((1, 8, 128), lambda b: (b, 0, 0)),
        out_shape=jax.ShapeDtypeStruct((B, 8, 128), jnp.int32),
    )(x)
    return out.reshape(B * KI)


def _temporal_body(noise_ref, pad_ref, fcst_ref, idx_ref, pm_ref):
    v = [noise_ref[m] for m in range(5)]
    pad = pad_ref[...]
    fcst = fcst_ref[...]
    ranks = []
    for m in range(5):
        rm = jnp.zeros((64, 128), jnp.int32)
        for mp in range(5):
            if mp == m:
                continue
            if mp < m:
                first = v[mp] <= v[m]
            else:
                first = v[mp] < v[m]
            rm = rm + first.astype(jnp.int32)
        ranks.append(rm)
    idx0 = jnp.zeros((64, 128), jnp.int32)
    idx1 = jnp.zeros((64, 128), jnp.int32)
    for m in range(5):
        idx0 = jnp.where(ranks[m] == 0, m, idx0)
        idx1 = jnp.where(ranks[m] == 1, m, idx1)
    r = lax.broadcasted_iota(jnp.int32, (64, 128), 0)
    c = lax.broadcasted_iota(jnp.int32, (64, 128), 1)
    f = r * 128 + c
    idx_ref[0] = f
    idx_ref[1] = (idx0 + 1) * (B * T) + f
    idx_ref[2] = (idx1 + 1) * (B * T) + f
    pm_ref[0] = pad
    pm_ref[1] = jnp.where(idx0 == 0, fcst, pad)
    pm_ref[2] = jnp.where(idx1 == 0, fcst, pad)


def _temporal_idx(noise_temporal, pad_mask, fcst_mask):
    noise5 = noise_temporal.transpose(2, 0, 1).reshape(5, 64, 128)
    pad = pad_mask.reshape(64, 128)
    fcst = fcst_mask.reshape(64, 128)
    tidx, tpm = pl.pallas_call(
        _temporal_body,
        out_shape=[
            jax.ShapeDtypeStruct((3, 64, 128), jnp.int32),
            jax.ShapeDtypeStruct((3, 64, 128), jnp.float32),
        ],
    )(noise5, pad, fcst)
    t_idx = tidx.reshape(-1)
    pm = tpm.reshape(3, B * T).T.reshape(B, T, 3)
    return t_idx, pm


def _regroup_body(i0, i1, i2, o):
    o[0, :, 0, :] = i0[0]
    o[0, :, 1, :] = i1[0]
    o[0, :, 2, :] = i2[0]


def _regroup(t_out):
    x = t_out.reshape(3, B * T, D)
    return pl.pallas_call(
        _regroup_body,
        grid=(64,),
        in_specs=[pl.BlockSpec((1, 128, D), lambda c, kk=k: (kk, c, 0))
                  for k in range(3)],
        out_specs=pl.BlockSpec((1, 128, 3, D), lambda c: (c // 8, c % 8, 0, 0)),
        out_shape=jax.ShapeDtypeStruct((B, T, 3, D), jnp.float32),
    )(x, x, x)


NT_ROWS = B * T * 3


def _sc_gather(t_tab, i_tab, n_tab, t_ix, i_ix, n_ix):
    mesh = plsc.VectorSubcoreMesh(core_axis_name="c", subcore_axis_name="s")

    @functools.partial(
        pl.kernel,
        mesh=mesh,
        out_type=[
            jax.ShapeDtypeStruct((NT_ROWS, D), jnp.float32),
            jax.ShapeDtypeStruct((B, 1025, D), jnp.float32),
            jax.ShapeDtypeStruct((B, 513, D), jnp.float32),
        ],
        scratch_types=[
            pltpu.VMEM((384,), jnp.int32),
            pltpu.VMEM((16,), jnp.int32),
            pltpu.VMEM((384, D), jnp.float32),
            pltpu.VMEM((16, D), jnp.float32),
            pltpu.SemaphoreType.DMA,
            pltpu.SemaphoreType.DMA,
        ],
    )
    def k(t_tab, i_tab, n_tab, t_ix, i_ix, n_ix, t_out, i_out, n_out,
          ixv, ixv16, rows, rows16, sem, sem2):
        wid = lax.axis_index("s") * 2 + lax.axis_index("c")
        b = wid // 4
        sub = wid % 4

        for ch in range(2):
            base = wid * 768 + ch * 384
            pltpu.sync_copy(t_ix.at[pl.ds(base, 384)], ixv)
            pltpu.async_copy(t_tab.at[ixv], rows, sem).wait()
            pltpu.sync_copy(rows, t_out.at[pl.ds(base, 384)])

        def chunk2(ix2, tab3, out3, base):
            pltpu.sync_copy(ix2.at[b, 0, pl.ds(base, 128)], ixv.at[pl.ds(0, 128)])
            pltpu.async_copy(tab3.at[b].at[ixv.at[pl.ds(0, 128)]],
                             rows.at[pl.ds(0, 128)], sem).wait()
            pltpu.sync_copy(rows.at[pl.ds(0, 128)], out3.at[b, pl.ds(base, 128)])

        def tail(ix2, tab3, out3, last):
            pltpu.sync_copy(ix2.at[b, 0, pl.ds(last, 16)], ixv16)
            pltpu.async_copy(tab3.at[b].at[ixv16], rows16, sem).wait()
            pltpu.sync_copy(rows16.at[pl.ds(0, 1)], out3.at[b, pl.ds(last, 1)])

        pltpu.sync_copy(i_ix.at[b, 0, pl.ds(sub * 256, 256)],
                        ixv.at[pl.ds(0, 256)])
        pltpu.async_copy(i_tab.at[b].at[ixv.at[pl.ds(0, 256)]],
                         rows.at[pl.ds(0, 256)], sem).wait()
        pltpu.sync_copy(rows.at[pl.ds(0, 256)], i_out.at[b, pl.ds(sub * 256, 256)])

        @pl.when(sub == 3)
        def _():
            tail(i_ix, i_tab, i_out, 1024)

        chunk2(n_ix, n_tab, n_out, sub * 128)

        @pl.when(sub == 2)
        def _():
            tail(n_ix, n_tab, n_out, 512)

    return k(t_tab, i_tab, n_tab, t_ix, i_ix, n_ix)


def kernel(temporal_data, img_data, nlp_data, temporal_padding_mask,
           target_fcst_mask, noise_temporal, noise_img, nlp_remain_idx):
    img_idx = _img_sort(noise_img).reshape(B, KI)
    t_idx, temporal_remain_pm = _temporal_idx(
        noise_temporal, temporal_padding_mask, target_fcst_mask)
    zb = jnp.zeros((B, 1), jnp.int32)
    i_ix = jnp.concatenate([zb, img_idx, jnp.zeros((B, 15), jnp.int32)],
                           axis=1).reshape(B, 1, 1040)
    n_src = nlp_remain_idx.astype(jnp.int32) + 1
    n_ix = jnp.concatenate([zb, n_src, jnp.zeros((B, 15), jnp.int32)],
                           axis=1).reshape(B, 1, 528)
    t_out, i_out, n_out = _sc_gather(
        temporal_data.reshape(M * B * T, D),
        img_data,
        nlp_data,
        t_idx,
        i_ix,
        n_ix,
    )
    temporal_block_remain = _regroup(t_out)
    img_remain = i_out
    nlp_remain = n_out
    img_remain_pm = jnp.ones((B, 1025), jnp.float32)
    return (temporal_block_remain, img_remain, nlp_remain,
            temporal_remain_pm, img_remain_pm)


# speedup vs baseline: 1.0598x; 1.0598x over previous
import functools

import jax
import jax.numpy as jnp
from jax import lax
from jax.experimental import pallas as pl
from jax.experimental.pallas import tpu as pltpu
from jax.experimental.pallas import tpu_sc as plsc

B, T, M, D = 8, 1024, 6, 256
NI = 4096
KI = 1024
NW = 32


def _img_sort_body(noise_ref, out_ref):
    keys = noise_ref[0]
    r = lax.broadcasted_iota(jnp.int32, (32, 128), 0)
    c = lax.broadcasted_iota(jnp.int32, (32, 128), 1)
    e = r * 128 + c
    idx = e
    k = 2
    while k <= NI:
        j = k // 2
        while j >= 1:
            if j < 128:
                ax, sh = 1, j
            else:
                ax, sh = 0, j // 128
            lower = (e & j) == 0
            pk = jnp.where(lower, jnp.roll(keys, -sh, axis=ax),
                           jnp.roll(keys, sh, axis=ax))
            pi = jnp.where(lower, jnp.roll(idx, -sh, axis=ax),
                           jnp.roll(idx, sh, axis=ax))
            a_first = (keys < pk) | ((keys == pk) & (idx < pi))
            want_small = lower == ((e & k) == 0)
            take_a = a_first == want_small
            keys = jnp.where(take_a, keys, pk)
            idx = jnp.where(take_a, idx, pi)
            j //= 2
        k *= 2
    out_ref[0] = idx[:8] + 1


def _img_sort(noise_img):
    x = noise_img.reshape(B, 32, 128)
    out = pl.pallas_call(
        _img_sort_body,
        grid=(B,),
        in_specs=[pl.BlockSpec((1, 32, 128), lambda b: (b, 0, 0))],
        out_specs=pl.BlockSpec((1, 8, 128), lambda b: (b, 0, 0)),
        out_shape=jax.ShapeDtypeStruct((B, 8, 128), jnp.int32),
    )(x)
    return out.reshape(B * KI)


def _temporal_body(noise_ref, pad_ref, fcst_ref, idx_ref, pm_ref):
    v = [noise_ref[m] for m in range(5)]
    pad = pad_ref[...]
    fcst = fcst_ref[...]
    ranks = []
    for m in range(5):
        rm = jnp.zeros((64, 128), jnp.int32)
        for mp in range(5):
            if mp == m:
                continue
            if mp < m:
                first = v[mp] <= v[m]
            else:
                first = v[mp] < v[m]
            rm = rm + first.astype(jnp.int32)
        ranks.append(rm)
    idx0 = jnp.zeros((64, 128), jnp.int32)
    idx1 = jnp.zeros((64, 128), jnp.int32)
    for m in range(5):
        idx0 = jnp.where(ranks[m] == 0, m, idx0)
        idx1 = jnp.where(ranks[m] == 1, m, idx1)
    r = lax.broadcasted_iota(jnp.int32, (64, 128), 0)
    c = lax.broadcasted_iota(jnp.int32, (64, 128), 1)
    f = r * 128 + c
    idx_ref[0] = f
    idx_ref[1] = (idx0 + 1) * (B * T) + f
    idx_ref[2] = (idx1 + 1) * (B * T) + f
    pm_ref[0] = pad
    pm_ref[1] = jnp.where(idx0 == 0, fcst, pad)
    pm_ref[2] = jnp.where(idx1 == 0, fcst, pad)


def _temporal_idx(noise_temporal, pad_mask, fcst_mask):
    noise5 = noise_temporal.transpose(2, 0, 1).reshape(5, 64, 128)
    pad = pad_mask.reshape(64, 128)
    fcst = fcst_mask.reshape(64, 128)
    tidx, tpm = pl.pallas_call(
        _temporal_body,
        out_shape=[
            jax.ShapeDtypeStruct((3, 64, 128), jnp.int32),
            jax.ShapeDtypeStruct((3, 64, 128), jnp.float32),
        ],
    )(noise5, pad, fcst)
    t_idx = tidx.reshape(3, B * T).T.reshape(-1)
    pm = tpm.reshape(3, B * T).T.reshape(B, T, 3)
    return t_idx, pm


NT_ROWS = B * T * 3


def _sc_gather(t_tab, i_tab, n_tab, t_ix, i_ix, n_ix):
    mesh = plsc.VectorSubcoreMesh(core_axis_name="c", subcore_axis_name="s")

    @functools.partial(
        pl.kernel,
        mesh=mesh,
        out_type=[
            jax.ShapeDtypeStruct((NT_ROWS, D), jnp.float32),
            jax.ShapeDtypeStruct((B, 1025, D), jnp.float32),
            jax.ShapeDtypeStruct((B, 513, D), jnp.float32),
        ],
        scratch_types=[
            pltpu.VMEM((384,), jnp.int32),
            pltpu.VMEM((16,), jnp.int32),
            pltpu.VMEM((384, D), jnp.float32),
            pltpu.VMEM((16, D), jnp.float32),
            pltpu.SemaphoreType.DMA,
            pltpu.SemaphoreType.DMA,
        ],
    )
    def k(t_tab, i_tab, n_tab, t_ix, i_ix, n_ix, t_out, i_out, n_out,
          ixv, ixv16, rows, rows16, sem, sem2):
        wid = lax.axis_index("s") * 2 + lax.axis_index("c")
        b = wid // 4
        sub = wid % 4

        for ch in range(2):
            base = wid * 768 + ch * 384
            pltpu.sync_copy(t_ix.at[pl.ds(base, 384)], ixv)
            pltpu.async_copy(t_tab.at[ixv], rows, sem).wait()
            pltpu.sync_copy(rows, t_out.at[pl.ds(base, 384)])

        def chunk2(ix2, tab3, out3, base):
            pltpu.sync_copy(ix2.at[b, 0, pl.ds(base, 128)], ixv.at[pl.ds(0, 128)])
            pltpu.async_copy(tab3.at[b].at[ixv.at[pl.ds(0, 128)]],
                             rows.at[pl.ds(0, 128)], sem).wait()
            pltpu.sync_copy(rows.at[pl.ds(0, 128)], out3.at[b, pl.ds(base, 128)])

        def tail(ix2, tab3, out3, last):
            pltpu.sync_copy(ix2.at[b, 0, pl.ds(last, 16)], ixv16)
            pltpu.async_copy(tab3.at[b].at[ixv16], rows16, sem).wait()
            pltpu.sync_copy(rows16.at[pl.ds(0, 1)], out3.at[b, pl.ds(last, 1)])

        pltpu.sync_copy(i_ix.at[b, 0, pl.ds(sub * 256, 256)],
                        ixv.at[pl.ds(0, 256)])
        pltpu.async_copy(i_tab.at[b].at[ixv.at[pl.ds(0, 256)]],
                         rows.at[pl.ds(0, 256)], sem).wait()
        pltpu.sync_copy(rows.at[pl.ds(0, 256)], i_out.at[b, pl.ds(sub * 256, 256)])

        @pl.when(sub == 3)
        def _():
            tail(i_ix, i_tab, i_out, 1024)

        chunk2(n_ix, n_tab, n_out, sub * 128)

        @pl.when(sub == 2)
        def _():
            tail(n_ix, n_tab, n_out, 512)

    return k(t_tab, i_tab, n_tab, t_ix, i_ix, n_ix)


def kernel(temporal_data, img_data, nlp_data, temporal_padding_mask,
           target_fcst_mask, noise_temporal, noise_img, nlp_remain_idx):
    img_idx = _img_sort(noise_img).reshape(B, KI)
    t_idx, temporal_remain_pm = _temporal_idx(
        noise_temporal, temporal_padding_mask, target_fcst_mask)
    zb = jnp.zeros((B, 1), jnp.int32)
    i_ix = jnp.concatenate([zb, img_idx, jnp.zeros((B, 15), jnp.int32)],
                           axis=1).reshape(B, 1, 1040)
    n_src = nlp_remain_idx.astype(jnp.int32) + 1
    n_ix = jnp.concatenate([zb, n_src, jnp.zeros((B, 15), jnp.int32)],
                           axis=1).reshape(B, 1, 528)
    t_out, i_out, n_out = _sc_gather(
        temporal_data.reshape(M * B * T, D),
        img_data,
        nlp_data,
        t_idx,
        i_ix,
        n_ix,
    )
    temporal_block_remain = t_out.reshape(B, T, 3, D)
    img_remain = i_out
    nlp_remain = n_out
    img_remain_pm = jnp.ones((B, 1025), jnp.float32)
    return (temporal_block_remain, img_remain, nlp_remain,
            temporal_remain_pm, img_remain_pm)
